# baseline (speedup 1.0000x reference)
import functools

import jax
import jax.numpy as jnp
from jax import lax
from jax.experimental import pallas as pl
from jax.experimental.pallas import tpu as pltpu
from jax.experimental.pallas import tpu_sc as plsc

_NC = 2
_NS = 16
_NW = _NC * _NS
_LANES = 16

_R = 128


def _make_sc_cumsum(b, t, d):
    dchunk = d // (_NW // b)
    ncg = dchunk // _LANES
    nblocks = t // _R
    chunks_per_batch = d // dchunk

    mesh = plsc.VectorSubcoreMesh(core_axis_name="c", subcore_axis_name="s")

    @functools.partial(
        pl.kernel,
        mesh=mesh,
        out_type=jax.ShapeDtypeStruct((b, t, d), jnp.float32),
        scratch_types=[
            pltpu.VMEM((_R, dchunk), jnp.float32),
            pltpu.VMEM((_R, dchunk), jnp.float32),
        ],
    )
    def sc_cumsum(x_hbm, out_hbm, inbuf, outbuf):
        wid = lax.axis_index("s") * _NC + lax.axis_index("c")
        bi = wid // chunks_per_batch
        d0 = (wid % chunks_per_batch) * dchunk

        def block_body(g, carries):
            r0 = g * _R
            pltpu.sync_copy(x_hbm.at[bi, pl.ds(r0, _R), pl.ds(d0, dchunk)],
                            inbuf)

            def row_body(i, cs):
                new = []
                for c in range(ncg):
                    v = inbuf[i, pl.ds(c * _LANES, _LANES)]
                    nv = cs[c] + v
                    outbuf[i, pl.ds(c * _LANES, _LANES)] = nv
                    new.append(nv)
                return tuple(new)

            carries = lax.fori_loop(0, _R, row_body, carries)
            pltpu.sync_copy(outbuf,
                            out_hbm.at[bi, pl.ds(r0, _R), pl.ds(d0, dchunk)])
            return carries

        zero = jnp.zeros((_LANES,), jnp.float32)
        lax.fori_loop(0, nblocks, block_body, (zero,) * ncg)

    return sc_cumsum


def kernel(x):
    b, t, d = x.shape
    out = _make_sc_cumsum(b, t, d)(x.astype(jnp.float32))
    return out.astype(x.dtype)
